# baseline (speedup 1.0000x reference)
import functools

import jax
import jax.numpy as jnp
from jax import lax
from jax.experimental import pallas as pl
from jax.experimental.pallas import tpu as pltpu
from jax.experimental.pallas import tpu_sc as plsc

N = 50000
E = 1600000
B = 10
NUM_CLASSES = 40

NC = 2
NS = 16
NW = NC * NS

NPAD = 51200
ROWS_T = NPAD // NS
ZR = 128
ZITERS = ROWS_T // ZR

KCH = 2
SLOTS = 3
RW0 = 630
RW1 = 942
ROWS_ALL = NS * (RW0 + RW1)
ED_PAD = ROWS_ALL * 128


def _make_segsum(W):
    mesh = plsc.VectorSubcoreMesh(core_axis_name="c", subcore_axis_name="s")

    @functools.partial(
        pl.kernel,
        mesh=mesh,
        compiler_params=pltpu.CompilerParams(use_tc_tiling_on_sc=False),
        out_type=jax.ShapeDtypeStruct((NC, NPAD, W), jnp.float32),
        scratch_types=[
            pltpu.VMEM((SLOTS, KCH, 2, 128), jnp.int32),
            pltpu.VMEM((SLOTS, KCH, 128, W), jnp.float32),
            pltpu.VMEM_SHARED((NPAD, W), jnp.float32),
            pltpu.SemaphoreType.DMA,
            pltpu.SemaphoreType.DMA,
            pltpu.SemaphoreType.DMA,
            pltpu.SemaphoreType.DMA,
            pltpu.SemaphoreType.DMA,
            pltpu.SemaphoreType.DMA,
        ],
    )
    def segsum(table, sdx, zsrc, out, idx, rows, acc,
               g0, g1, g2, s0, s1, s2):
        gsem = (g0, g1, g2)
        ssem = (s0, s1, s2)
        cid = lax.axis_index("c")
        sid = lax.axis_index("s")
        wid = sid * NC + cid

        pltpu.sync_copy(zsrc, rows.at[0, 0])

        def zero_chunk(i, carry):
            pltpu.sync_copy(rows.at[0, 0],
                            acc.at[pl.ds(sid * ROWS_T + i * ZR, ZR)])
            return carry

        lax.fori_loop(0, ZITERS, zero_chunk, 0)
        plsc.subcore_barrier()

        base = jnp.where(cid == 0, sid * RW0, NS * RW0 + sid * RW1)
        nrounds = jnp.where(cid == 0, RW0 // KCH // SLOTS, RW1 // KCH // SLOTS)

        def fire_gathers(b, k):
            pltpu.sync_copy(sdx.at[pl.ds(base + k * KCH, KCH)], idx.at[b])
            for j in range(KCH):
                pltpu.async_copy(table.at[idx.at[b, j, 0]], rows.at[b, j],
                                 gsem[b])

        def wait_gathers(b):
            for j in range(KCH):
                pltpu.make_async_copy(table.at[idx.at[b, j, 0]],
                                      rows.at[b, j], gsem[b]).wait()

        def fire_scatters(b):
            for j in range(KCH):
                pltpu.async_copy(rows.at[b, j], acc.at[idx.at[b, j, 1]],
                                 ssem[b], add=True)

        def wait_scatters(b):
            for j in range(KCH):
                pltpu.make_async_copy(rows.at[b, j], acc.at[idx.at[b, j, 1]],
                                      ssem[b]).wait()

        for b in range(SLOTS):
            fire_gathers(b, b)

        def ring_round(t, carry):
            for b in range(SLOTS):
                wait_gathers(b)
                fire_scatters(b)
            for b in range(SLOTS):
                wait_scatters(b)
                fire_gathers(b, SLOTS * t + SLOTS + b)
            return carry

        lax.fori_loop(0, nrounds - 1, ring_round, 0)
        for b in range(SLOTS):
            wait_gathers(b)
            fire_scatters(b)
        for b in range(SLOTS):
            wait_scatters(b)
        plsc.subcore_barrier()

        def out_chunk(i, carry):
            r = sid * ROWS_T + i * ZR
            pltpu.sync_copy(acc.at[pl.ds(r, ZR)], rows.at[0, 0])
            pltpu.sync_copy(rows.at[0, 0], out.at[cid].at[pl.ds(r, ZR)])
            return carry

        lax.fori_loop(0, ZITERS, out_chunk, 0)

    return segsum


_segsum8 = _make_segsum(8)
_segsum32 = _make_segsum(32)


R1 = 3200
G1 = NPAD // R1


def _tc1_body(verts_ref, lrf_ref, acc_ref, w1s_ref, w1n_ref, b1_ref, out_ref):
    i = pl.program_id(0)
    s = acc_ref[0] + acc_ref[1]
    deg = s[:, 3:4]
    v = verts_ref[...]
    l9 = lrf_ref[...]
    rel = [s[:, j:j + 1] - deg * v[:, j:j + 1] for j in range(3)]
    out = b1_ref[...]
    for k in range(3):
        out = out + v[:, k:k + 1] * w1s_ref[k:k + 1, :]
    for r in range(3):
        rot_r = sum(l9[:, 3 * r + j:3 * r + j + 1] * rel[j] for j in range(3))
        out = out + rot_r * w1n_ref[r:r + 1, :]
    out = jnp.maximum(out, 0.0)
    rid = i * R1 + lax.broadcasted_iota(jnp.int32, (R1, 32), 0)
    out_ref[...] = jnp.where(rid < N, out, 0.0)


_tc1 = pl.pallas_call(
    _tc1_body,
    grid=(G1,),
    in_specs=[
        pl.BlockSpec((R1, 3), lambda i: (i, 0)),
        pl.BlockSpec((R1, 9), lambda i: (i, 0)),
        pl.BlockSpec((2, R1, 8), lambda i: (0, i, 0)),
        pl.BlockSpec((3, 32), lambda i: (0, 0)),
        pl.BlockSpec((3, 32), lambda i: (0, 0)),
        pl.BlockSpec((1, 32), lambda i: (0, 0)),
    ],
    out_specs=pl.BlockSpec((R1, 32), lambda i: (i, 0)),
    out_shape=jax.ShapeDtypeStruct((NPAD, 32), jnp.float32),
)


R2 = 1000
G2 = N // R2
BLK_PER_SEG = (N // B) // R2


def _tc2_body(out1_ref, acc_ref, w2s_ref, w2n_ref, b2_ref,
              wc1_ref, bc1_ref, wc2_ref, bc2_ref, out_ref, mref):
    i = pl.program_id(0)
    g = acc_ref[0] + acc_ref[1]
    h = jnp.dot(out1_ref[...], w2s_ref[...], preferred_element_type=jnp.float32)
    h = h + jnp.dot(g, w2n_ref[...], preferred_element_type=jnp.float32)
    out2 = jnp.maximum(h + b2_ref[...], 0.0)
    m = jnp.max(out2, axis=0, keepdims=True)
    seg = i // BLK_PER_SEG

    @pl.when(i % BLK_PER_SEG == 0)
    def _():
        mref[pl.ds(seg, 1), :] = m

    @pl.when(i % BLK_PER_SEG != 0)
    def _():
        mref[pl.ds(seg, 1), :] = jnp.maximum(mref[pl.ds(seg, 1), :], m)

    @pl.when(i == G2 - 1)
    def _():
        mf = mref[...]
        hc = jnp.dot(mf, wc1_ref[...], preferred_element_type=jnp.float32)
        hc = jnp.maximum(hc + bc1_ref[...], 0.0)
        logits = jnp.dot(hc, wc2_ref[...], preferred_element_type=jnp.float32)
        out_ref[...] = logits + bc2_ref[...]


_tc2 = pl.pallas_call(
    _tc2_body,
    grid=(G2,),
    in_specs=[
        pl.BlockSpec((R2, 32), lambda i: (i, 0)),
        pl.BlockSpec((2, R2, 32), lambda i: (0, i, 0)),
        pl.BlockSpec((32, 128), lambda i: (0, 0)),
        pl.BlockSpec((32, 128), lambda i: (0, 0)),
        pl.BlockSpec((1, 128), lambda i: (0, 0)),
        pl.BlockSpec((128, 64), lambda i: (0, 0)),
        pl.BlockSpec((1, 64), lambda i: (0, 0)),
        pl.BlockSpec((64, NUM_CLASSES), lambda i: (0, 0)),
        pl.BlockSpec((1, NUM_CLASSES), lambda i: (0, 0)),
    ],
    out_specs=pl.BlockSpec((B, NUM_CLASSES), lambda i: (0, 0)),
    out_shape=jax.ShapeDtypeStruct((B, NUM_CLASSES), jnp.float32),
    scratch_shapes=[pltpu.VMEM((B, 128), jnp.float32)],
)


def kernel(verts, edges, lrf, vert_num_list, W1_self, b1, W1_nbr,
           W2_self, b2, W2_nbr, Wc1, bc1, Wc2, bc2):
    e0 = edges[:, 0]
    e1 = edges[:, 1]
    pad = jnp.full((ED_PAD - 2 * E,), N, dtype=jnp.int32)
    src2 = jnp.concatenate([e0, e1, pad]).reshape(-1, 128)
    dst2 = jnp.concatenate([e1, e0, pad]).reshape(-1, 128)
    sdx = jnp.stack([src2, dst2], axis=1)

    verts8 = jnp.zeros((NPAD, 8), jnp.float32)
    verts8 = verts8.at[:N, :3].set(verts).at[:N, 3].set(1.0)
    zsrc8 = jnp.zeros((ZR, 8), jnp.float32)
    zsrc32 = jnp.zeros((ZR, 32), jnp.float32)

    acc_a = _segsum8(verts8, sdx, zsrc8)

    verts_p = jnp.zeros((NPAD, 3), jnp.float32).at[:N].set(verts)
    lrf9_p = jnp.zeros((NPAD, 9), jnp.float32).at[:N].set(lrf.reshape(N, 9))
    out1 = _tc1(verts_p, lrf9_p, acc_a, W1_self, W1_nbr,
                b1.reshape(1, 32))

    acc_b = _segsum32(out1, sdx, zsrc32)

    logits = _tc2(out1, acc_b, W2_self, W2_nbr, b2.reshape(1, 128),
                  Wc1, bc1.reshape(1, 64), Wc2, bc2.reshape(1, NUM_CLASSES))
    return logits


# speedup vs baseline: 1.0931x; 1.0931x over previous
import functools

import jax
import jax.numpy as jnp
from jax import lax
from jax.experimental import pallas as pl
from jax.experimental.pallas import tpu as pltpu
from jax.experimental.pallas import tpu_sc as plsc

N = 50000
E = 1600000
B = 10
NUM_CLASSES = 40

NC = 2
NS = 16
NW = NC * NS

NPAD = 51200
ROWS_T = NPAD // NS
ZR = 128
ZITERS = ROWS_T // ZR

KCH = 2
SLOTS = 3
RW0 = 942
RW1 = 630
ROWS_ALL = NS * (RW0 + RW1)
ED_PAD = ROWS_ALL * 128


def _make_segsum(W):
    mesh = plsc.VectorSubcoreMesh(core_axis_name="c", subcore_axis_name="s")

    @functools.partial(
        pl.kernel,
        mesh=mesh,
        compiler_params=pltpu.CompilerParams(use_tc_tiling_on_sc=False),
        out_type=jax.ShapeDtypeStruct((NC, NPAD, W), jnp.float32),
        scratch_types=[
            pltpu.VMEM((SLOTS, KCH, 2, 128), jnp.int32),
            pltpu.VMEM((SLOTS, KCH, 128, W), jnp.float32),
            pltpu.VMEM_SHARED((NPAD, W), jnp.float32),
            pltpu.SemaphoreType.DMA,
            pltpu.SemaphoreType.DMA,
            pltpu.SemaphoreType.DMA,
            pltpu.SemaphoreType.DMA,
            pltpu.SemaphoreType.DMA,
            pltpu.SemaphoreType.DMA,
        ],
    )
    def segsum(table, sdx, zsrc, out, idx, rows, acc,
               g0, g1, g2, s0, s1, s2):
        gsem = (g0, g1, g2)
        ssem = (s0, s1, s2)
        cid = lax.axis_index("c")
        sid = lax.axis_index("s")
        wid = sid * NC + cid

        pltpu.sync_copy(zsrc, rows.at[0, 0])

        def zero_chunk(i, carry):
            pltpu.sync_copy(rows.at[0, 0],
                            acc.at[pl.ds(sid * ROWS_T + i * ZR, ZR)])
            return carry

        lax.fori_loop(0, ZITERS, zero_chunk, 0)
        plsc.subcore_barrier()

        base = jnp.where(cid == 0, sid * RW0, NS * RW0 + sid * RW1)
        nrounds = jnp.where(cid == 0, RW0 // KCH // SLOTS, RW1 // KCH // SLOTS)

        def fire_gathers(b, k):
            pltpu.sync_copy(sdx.at[pl.ds(base + k * KCH, KCH)], idx.at[b])
            for j in range(KCH):
                pltpu.async_copy(table.at[idx.at[b, j, 0]], rows.at[b, j],
                                 gsem[b])

        def wait_gathers(b):
            for j in range(KCH):
                pltpu.make_async_copy(table.at[idx.at[b, j, 0]],
                                      rows.at[b, j], gsem[b]).wait()

        def fire_scatters(b):
            for j in range(KCH):
                pltpu.async_copy(rows.at[b, j], acc.at[idx.at[b, j, 1]],
                                 ssem[b], add=True)

        def wait_scatters(b):
            for j in range(KCH):
                pltpu.make_async_copy(rows.at[b, j], acc.at[idx.at[b, j, 1]],
                                      ssem[b]).wait()

        for b in range(SLOTS):
            fire_gathers(b, b)

        def ring_round(t, carry):
            for b in range(SLOTS):
                wait_gathers(b)
                fire_scatters(b)
            for b in range(SLOTS):
                wait_scatters(b)
                fire_gathers(b, SLOTS * t + SLOTS + b)
            return carry

        lax.fori_loop(0, nrounds - 1, ring_round, 0)
        for b in range(SLOTS):
            wait_gathers(b)
            fire_scatters(b)
        for b in range(SLOTS):
            wait_scatters(b)
        plsc.subcore_barrier()

        def out_chunk(i, carry):
            r = sid * ROWS_T + i * ZR
            pltpu.sync_copy(acc.at[pl.ds(r, ZR)], rows.at[0, 0])
            pltpu.sync_copy(rows.at[0, 0], out.at[cid].at[pl.ds(r, ZR)])
            return carry

        lax.fori_loop(0, ZITERS, out_chunk, 0)

    return segsum


_segsum8 = _make_segsum(8)
_segsum32 = _make_segsum(32)


R1 = 3200
G1 = NPAD // R1


def _tc1_body(verts_ref, lrf_ref, acc_ref, w1s_ref, w1n_ref, b1_ref, out_ref):
    i = pl.program_id(0)
    s = acc_ref[0] + acc_ref[1]
    deg = s[:, 3:4]
    v = verts_ref[...]
    l9 = lrf_ref[...]
    rel = [s[:, j:j + 1] - deg * v[:, j:j + 1] for j in range(3)]
    out = b1_ref[...]
    for k in range(3):
        out = out + v[:, k:k + 1] * w1s_ref[k:k + 1, :]
    for r in range(3):
        rot_r = sum(l9[:, 3 * r + j:3 * r + j + 1] * rel[j] for j in range(3))
        out = out + rot_r * w1n_ref[r:r + 1, :]
    out = jnp.maximum(out, 0.0)
    rid = i * R1 + lax.broadcasted_iota(jnp.int32, (R1, 32), 0)
    out_ref[...] = jnp.where(rid < N, out, 0.0)


_tc1 = pl.pallas_call(
    _tc1_body,
    grid=(G1,),
    in_specs=[
        pl.BlockSpec((R1, 3), lambda i: (i, 0)),
        pl.BlockSpec((R1, 9), lambda i: (i, 0)),
        pl.BlockSpec((2, R1, 8), lambda i: (0, i, 0)),
        pl.BlockSpec((3, 32), lambda i: (0, 0)),
        pl.BlockSpec((3, 32), lambda i: (0, 0)),
        pl.BlockSpec((1, 32), lambda i: (0, 0)),
    ],
    out_specs=pl.BlockSpec((R1, 32), lambda i: (i, 0)),
    out_shape=jax.ShapeDtypeStruct((NPAD, 32), jnp.float32),
)


R2 = 1000
G2 = N // R2
BLK_PER_SEG = (N // B) // R2


def _tc2_body(out1_ref, acc_ref, w2s_ref, w2n_ref, b2_ref,
              wc1_ref, bc1_ref, wc2_ref, bc2_ref, out_ref, mref):
    i = pl.program_id(0)
    g = acc_ref[0] + acc_ref[1]
    h = jnp.dot(out1_ref[...], w2s_ref[...], preferred_element_type=jnp.float32)
    h = h + jnp.dot(g, w2n_ref[...], preferred_element_type=jnp.float32)
    out2 = jnp.maximum(h + b2_ref[...], 0.0)
    m = jnp.max(out2, axis=0, keepdims=True)
    seg = i // BLK_PER_SEG

    @pl.when(i % BLK_PER_SEG == 0)
    def _():
        mref[pl.ds(seg, 1), :] = m

    @pl.when(i % BLK_PER_SEG != 0)
    def _():
        mref[pl.ds(seg, 1), :] = jnp.maximum(mref[pl.ds(seg, 1), :], m)

    @pl.when(i == G2 - 1)
    def _():
        mf = mref[...]
        hc = jnp.dot(mf, wc1_ref[...], preferred_element_type=jnp.float32)
        hc = jnp.maximum(hc + bc1_ref[...], 0.0)
        logits = jnp.dot(hc, wc2_ref[...], preferred_element_type=jnp.float32)
        out_ref[...] = logits + bc2_ref[...]


_tc2 = pl.pallas_call(
    _tc2_body,
    grid=(G2,),
    in_specs=[
        pl.BlockSpec((R2, 32), lambda i: (i, 0)),
        pl.BlockSpec((2, R2, 32), lambda i: (0, i, 0)),
        pl.BlockSpec((32, 128), lambda i: (0, 0)),
        pl.BlockSpec((32, 128), lambda i: (0, 0)),
        pl.BlockSpec((1, 128), lambda i: (0, 0)),
        pl.BlockSpec((128, 64), lambda i: (0, 0)),
        pl.BlockSpec((1, 64), lambda i: (0, 0)),
        pl.BlockSpec((64, NUM_CLASSES), lambda i: (0, 0)),
        pl.BlockSpec((1, NUM_CLASSES), lambda i: (0, 0)),
    ],
    out_specs=pl.BlockSpec((B, NUM_CLASSES), lambda i: (0, 0)),
    out_shape=jax.ShapeDtypeStruct((B, NUM_CLASSES), jnp.float32),
    scratch_shapes=[pltpu.VMEM((B, 128), jnp.float32)],
)


def kernel(verts, edges, lrf, vert_num_list, W1_self, b1, W1_nbr,
           W2_self, b2, W2_nbr, Wc1, bc1, Wc2, bc2):
    e0 = edges[:, 0]
    e1 = edges[:, 1]
    pad = jnp.full((ED_PAD - 2 * E,), N, dtype=jnp.int32)
    src2 = jnp.concatenate([e0, e1, pad]).reshape(-1, 128)
    dst2 = jnp.concatenate([e1, e0, pad]).reshape(-1, 128)
    sdx = jnp.stack([src2, dst2], axis=1)

    verts8 = jnp.zeros((NPAD, 8), jnp.float32)
    verts8 = verts8.at[:N, :3].set(verts).at[:N, 3].set(1.0)
    zsrc8 = jnp.zeros((ZR, 8), jnp.float32)
    zsrc32 = jnp.zeros((ZR, 32), jnp.float32)

    acc_a = _segsum8(verts8, sdx, zsrc8)

    verts_p = jnp.zeros((NPAD, 3), jnp.float32).at[:N].set(verts)
    lrf9_p = jnp.zeros((NPAD, 9), jnp.float32).at[:N].set(lrf.reshape(N, 9))
    out1 = _tc1(verts_p, lrf9_p, acc_a, W1_self, W1_nbr,
                b1.reshape(1, 32))

    acc_b = _segsum32(out1, sdx, zsrc32)

    logits = _tc2(out1, acc_b, W2_self, W2_nbr, b2.reshape(1, 128),
                  Wc1, bc1.reshape(1, 64), Wc2, bc2.reshape(1, NUM_CLASSES))
    return logits


# speedup vs baseline: 1.1173x; 1.0222x over previous
import functools

import jax
import jax.numpy as jnp
from jax import lax
from jax.experimental import pallas as pl
from jax.experimental.pallas import tpu as pltpu
from jax.experimental.pallas import tpu_sc as plsc

N = 50000
E = 1600000
B = 10
NUM_CLASSES = 40

NC = 2
NS = 16
NW = NC * NS

NPAD = 51200
ROWS_T = NPAD // NS
ZR = 128
ZITERS = ROWS_T // ZR

KCH = 2
SLOTS = 3
RW_SUM = 1572
ROWS_ALL = NS * RW_SUM
ED_PAD = ROWS_ALL * 128


def _make_segsum(W, RW0, RW1):
    mesh = plsc.VectorSubcoreMesh(core_axis_name="c", subcore_axis_name="s")

    @functools.partial(
        pl.kernel,
        mesh=mesh,
        compiler_params=pltpu.CompilerParams(use_tc_tiling_on_sc=False),
        out_type=jax.ShapeDtypeStruct((NC, NPAD, W), jnp.float32),
        scratch_types=[
            pltpu.VMEM((SLOTS, KCH, 2, 128), jnp.int32),
            pltpu.VMEM((SLOTS, KCH, 128, W), jnp.float32),
            pltpu.VMEM_SHARED((NPAD, W), jnp.float32),
            pltpu.SemaphoreType.DMA,
            pltpu.SemaphoreType.DMA,
            pltpu.SemaphoreType.DMA,
            pltpu.SemaphoreType.DMA,
            pltpu.SemaphoreType.DMA,
            pltpu.SemaphoreType.DMA,
        ],
    )
    def segsum(table, sdx, zsrc, out, idx, rows, acc,
               g0, g1, g2, s0, s1, s2):
        gsem = (g0, g1, g2)
        ssem = (s0, s1, s2)
        cid = lax.axis_index("c")
        sid = lax.axis_index("s")
        wid = sid * NC + cid

        pltpu.sync_copy(zsrc, rows.at[0, 0])

        def zero_chunk(i, carry):
            pltpu.sync_copy(rows.at[0, 0],
                            acc.at[pl.ds(sid * ROWS_T + i * ZR, ZR)])
            return carry

        lax.fori_loop(0, ZITERS, zero_chunk, 0)
        plsc.subcore_barrier()

        base = jnp.where(cid == 0, sid * RW0, NS * RW0 + sid * RW1)
        nrounds = jnp.where(cid == 0, RW0 // KCH // SLOTS, RW1 // KCH // SLOTS)

        def fire_gathers(b, k):
            pltpu.sync_copy(sdx.at[pl.ds(base + k * KCH, KCH)], idx.at[b])
            for j in range(KCH):
                pltpu.async_copy(table.at[idx.at[b, j, 0]], rows.at[b, j],
                                 gsem[b])

        def wait_gathers(b):
            for j in range(KCH):
                pltpu.make_async_copy(table.at[idx.at[b, j, 0]],
                                      rows.at[b, j], gsem[b]).wait()

        def fire_scatters(b):
            for j in range(KCH):
                pltpu.async_copy(rows.at[b, j], acc.at[idx.at[b, j, 1]],
                                 ssem[b], add=True)

        def wait_scatters(b):
            for j in range(KCH):
                pltpu.make_async_copy(rows.at[b, j], acc.at[idx.at[b, j, 1]],
                                      ssem[b]).wait()

        for b in range(SLOTS):
            fire_gathers(b, b)

        def ring_round(t, carry):
            for b in range(SLOTS):
                wait_gathers(b)
                fire_scatters(b)
            for b in range(SLOTS):
                wait_scatters(b)
                fire_gathers(b, SLOTS * t + SLOTS + b)
            return carry

        lax.fori_loop(0, nrounds - 1, ring_round, 0)
        for b in range(SLOTS):
            wait_gathers(b)
            fire_scatters(b)
        for b in range(SLOTS):
            wait_scatters(b)
        plsc.subcore_barrier()

        def out_chunk(i, carry):
            r = sid * ROWS_T + i * ZR
            pltpu.sync_copy(acc.at[pl.ds(r, ZR)], rows.at[0, 0])
            pltpu.sync_copy(rows.at[0, 0], out.at[cid].at[pl.ds(r, ZR)])
            return carry

        lax.fori_loop(0, ZITERS, out_chunk, 0)

    return segsum


_segsum8 = _make_segsum(8, 852, 720)
_segsum32 = _make_segsum(32, 924, 648)


R1 = 3200
G1 = NPAD // R1


def _tc1_body(verts_ref, lrf_ref, acc_ref, w1s_ref, w1n_ref, b1_ref, out_ref):
    i = pl.program_id(0)
    s = acc_ref[0] + acc_ref[1]
    deg = s[:, 3:4]
    v = verts_ref[...]
    l9 = lrf_ref[...]
    rel = [s[:, j:j + 1] - deg * v[:, j:j + 1] for j in range(3)]
    out = b1_ref[...]
    for k in range(3):
        out = out + v[:, k:k + 1] * w1s_ref[k:k + 1, :]
    for r in range(3):
        rot_r = sum(l9[:, 3 * r + j:3 * r + j + 1] * rel[j] for j in range(3))
        out = out + rot_r * w1n_ref[r:r + 1, :]
    out = jnp.maximum(out, 0.0)
    rid = i * R1 + lax.broadcasted_iota(jnp.int32, (R1, 32), 0)
    out_ref[...] = jnp.where(rid < N, out, 0.0)


_tc1 = pl.pallas_call(
    _tc1_body,
    grid=(G1,),
    in_specs=[
        pl.BlockSpec((R1, 3), lambda i: (i, 0)),
        pl.BlockSpec((R1, 9), lambda i: (i, 0)),
        pl.BlockSpec((2, R1, 8), lambda i: (0, i, 0)),
        pl.BlockSpec((3, 32), lambda i: (0, 0)),
        pl.BlockSpec((3, 32), lambda i: (0, 0)),
        pl.BlockSpec((1, 32), lambda i: (0, 0)),
    ],
    out_specs=pl.BlockSpec((R1, 32), lambda i: (i, 0)),
    out_shape=jax.ShapeDtypeStruct((NPAD, 32), jnp.float32),
)


R2 = 1000
G2 = N // R2
BLK_PER_SEG = (N // B) // R2


def _tc2_body(out1_ref, acc_ref, w2s_ref, w2n_ref, b2_ref,
              wc1_ref, bc1_ref, wc2_ref, bc2_ref, out_ref, mref):
    i = pl.program_id(0)
    g = acc_ref[0] + acc_ref[1]
    h = jnp.dot(out1_ref[...], w2s_ref[...], preferred_element_type=jnp.float32)
    h = h + jnp.dot(g, w2n_ref[...], preferred_element_type=jnp.float32)
    out2 = jnp.maximum(h + b2_ref[...], 0.0)
    m = jnp.max(out2, axis=0, keepdims=True)
    seg = i // BLK_PER_SEG

    @pl.when(i % BLK_PER_SEG == 0)
    def _():
        mref[pl.ds(seg, 1), :] = m

    @pl.when(i % BLK_PER_SEG != 0)
    def _():
        mref[pl.ds(seg, 1), :] = jnp.maximum(mref[pl.ds(seg, 1), :], m)

    @pl.when(i == G2 - 1)
    def _():
        mf = mref[...]
        hc = jnp.dot(mf, wc1_ref[...], preferred_element_type=jnp.float32)
        hc = jnp.maximum(hc + bc1_ref[...], 0.0)
        logits = jnp.dot(hc, wc2_ref[...], preferred_element_type=jnp.float32)
        out_ref[...] = logits + bc2_ref[...]


_tc2 = pl.pallas_call(
    _tc2_body,
    grid=(G2,),
    in_specs=[
        pl.BlockSpec((R2, 32), lambda i: (i, 0)),
        pl.BlockSpec((2, R2, 32), lambda i: (0, i, 0)),
        pl.BlockSpec((32, 128), lambda i: (0, 0)),
        pl.BlockSpec((32, 128), lambda i: (0, 0)),
        pl.BlockSpec((1, 128), lambda i: (0, 0)),
        pl.BlockSpec((128, 64), lambda i: (0, 0)),
        pl.BlockSpec((1, 64), lambda i: (0, 0)),
        pl.BlockSpec((64, NUM_CLASSES), lambda i: (0, 0)),
        pl.BlockSpec((1, NUM_CLASSES), lambda i: (0, 0)),
    ],
    out_specs=pl.BlockSpec((B, NUM_CLASSES), lambda i: (0, 0)),
    out_shape=jax.ShapeDtypeStruct((B, NUM_CLASSES), jnp.float32),
    scratch_shapes=[pltpu.VMEM((B, 128), jnp.float32)],
)


def kernel(verts, edges, lrf, vert_num_list, W1_self, b1, W1_nbr,
           W2_self, b2, W2_nbr, Wc1, bc1, Wc2, bc2):
    e0 = edges[:, 0]
    e1 = edges[:, 1]
    pad = jnp.full((ED_PAD - 2 * E,), N, dtype=jnp.int32)
    src2 = jnp.concatenate([e0, e1, pad]).reshape(-1, 128)
    dst2 = jnp.concatenate([e1, e0, pad]).reshape(-1, 128)
    sdx = jnp.stack([src2, dst2], axis=1)

    verts8 = jnp.zeros((NPAD, 8), jnp.float32)
    verts8 = verts8.at[:N, :3].set(verts).at[:N, 3].set(1.0)
    zsrc8 = jnp.zeros((ZR, 8), jnp.float32)
    zsrc32 = jnp.zeros((ZR, 32), jnp.float32)

    acc_a = _segsum8(verts8, sdx, zsrc8)

    verts_p = jnp.zeros((NPAD, 3), jnp.float32).at[:N].set(verts)
    lrf9_p = jnp.zeros((NPAD, 9), jnp.float32).at[:N].set(lrf.reshape(N, 9))
    out1 = _tc1(verts_p, lrf9_p, acc_a, W1_self, W1_nbr,
                b1.reshape(1, 32))

    acc_b = _segsum32(out1, sdx, zsrc32)

    logits = _tc2(out1, acc_b, W2_self, W2_nbr, b2.reshape(1, 128),
                  Wc1, bc1.reshape(1, 64), Wc2, bc2.reshape(1, NUM_CLASSES))
    return logits


# speedup vs baseline: 1.2462x; 1.1154x over previous
import functools

import jax
import jax.numpy as jnp
from jax import lax
from jax.experimental import pallas as pl
from jax.experimental.pallas import tpu as pltpu
from jax.experimental.pallas import tpu_sc as plsc

N = 50000
E = 1600000
B = 10
NUM_CLASSES = 40

NC = 2
NS = 16
NW = NC * NS

NPAD = 51200
ROWS_T = NPAD // NS
ZR = 128
ZITERS = ROWS_T // ZR

KCH = 2
SLOTS = 3
RW_SUM = 1572
ROWS_ALL = NS * RW_SUM
ED_PAD = ROWS_ALL * 128


def _make_segsum(W, RW0, RW1):
    mesh = plsc.VectorSubcoreMesh(core_axis_name="c", subcore_axis_name="s")

    @functools.partial(
        pl.kernel,
        mesh=mesh,
        compiler_params=pltpu.CompilerParams(use_tc_tiling_on_sc=False),
        out_type=jax.ShapeDtypeStruct((NC, NPAD, W), jnp.float32),
        scratch_types=[
            pltpu.VMEM((SLOTS, KCH, 2, 128), jnp.int32),
            pltpu.VMEM((SLOTS, KCH, 128, W), jnp.float32),
            pltpu.VMEM_SHARED((NPAD, W), jnp.float32),
            pltpu.SemaphoreType.DMA,
            pltpu.SemaphoreType.DMA,
            pltpu.SemaphoreType.DMA,
            pltpu.SemaphoreType.DMA,
            pltpu.SemaphoreType.DMA,
            pltpu.SemaphoreType.DMA,
        ],
    )
    def segsum(table, sdx, zsrc, out, idx, rows, acc,
               g0, g1, g2, s0, s1, s2):
        gsem = (g0, g1, g2)
        ssem = (s0, s1, s2)
        cid = lax.axis_index("c")
        sid = lax.axis_index("s")
        wid = sid * NC + cid

        pltpu.sync_copy(zsrc, rows.at[0, 0])

        def zero_chunk(i, carry):
            pltpu.sync_copy(rows.at[0, 0],
                            acc.at[pl.ds(sid * ROWS_T + i * ZR, ZR)])
            return carry

        lax.fori_loop(0, ZITERS, zero_chunk, 0)
        plsc.subcore_barrier()

        base = jnp.where(cid == 0, sid * RW0, NS * RW0 + sid * RW1)
        nrounds = jnp.where(cid == 0, RW0 // KCH // SLOTS, RW1 // KCH // SLOTS)

        def fire_gathers(b, k):
            pltpu.sync_copy(sdx.at[pl.ds(base + k * KCH, KCH)], idx.at[b])
            for j in range(KCH):
                pltpu.async_copy(table.at[idx.at[b, j, 0]], rows.at[b, j],
                                 gsem[b])

        def wait_gathers(b):
            for j in range(KCH):
                pltpu.make_async_copy(table.at[idx.at[b, j, 0]],
                                      rows.at[b, j], gsem[b]).wait()

        def fire_scatters(b):
            for j in range(KCH):
                pltpu.async_copy(rows.at[b, j], acc.at[idx.at[b, j, 1]],
                                 ssem[b], add=True)

        def wait_scatters(b):
            for j in range(KCH):
                pltpu.make_async_copy(rows.at[b, j], acc.at[idx.at[b, j, 1]],
                                      ssem[b]).wait()

        for b in range(SLOTS):
            fire_gathers(b, b)

        def ring_round(t, carry):
            for b in range(SLOTS):
                wait_gathers(b)
                fire_scatters(b)
            for b in range(SLOTS):
                wait_scatters(b)
                fire_gathers(b, SLOTS * t + SLOTS + b)
            return carry

        lax.fori_loop(0, nrounds - 1, ring_round, 0)
        for b in range(SLOTS):
            wait_gathers(b)
            fire_scatters(b)
        for b in range(SLOTS):
            wait_scatters(b)
        plsc.subcore_barrier()

        def out_chunk(i, carry):
            r = sid * ROWS_T + i * ZR
            pltpu.sync_copy(acc.at[pl.ds(r, ZR)], rows.at[0, 0])
            pltpu.sync_copy(rows.at[0, 0], out.at[cid].at[pl.ds(r, ZR)])
            return carry

        lax.fori_loop(0, ZITERS, out_chunk, 0)

    return segsum


_segsum8 = _make_segsum(8, 852, 720)
_segsum32 = _make_segsum(32, 924, 648)


CHV = NPAD // NW
PCV = 400
NCHV = CHV // PCV
UNR = 4


def _make_vert():
    mesh = plsc.VectorSubcoreMesh(core_axis_name="c", subcore_axis_name="s")

    @functools.partial(
        pl.kernel,
        mesh=mesh,
        compiler_params=pltpu.CompilerParams(
            use_tc_tiling_on_sc=False, needs_layout_passes=False),
        out_type=jax.ShapeDtypeStruct((NPAD, 32), jnp.float32),
        scratch_types=[
            pltpu.VMEM((PCV, 8), jnp.float32),
            pltpu.VMEM((PCV, 8), jnp.float32),
            pltpu.VMEM((PCV, 8), jnp.float32),
            pltpu.VMEM((PCV, 9), jnp.float32),
            pltpu.VMEM((PCV, 32), jnp.float32),
            pltpu.VMEM((3, 32), jnp.float32),
            pltpu.VMEM((3, 32), jnp.float32),
            pltpu.VMEM((32,), jnp.float32),
        ],
    )
    def vert(acc_a, v8, l9, w1s, w1n, b1, out1,
             acc0b, acc1b, vbuf, lbuf, obuf, wsb, wnb, bb):
        cid = lax.axis_index("c")
        sid = lax.axis_index("s")
        wid = sid * NC + cid
        base = wid * CHV

        pltpu.sync_copy(w1s, wsb)
        pltpu.sync_copy(w1n, wnb)
        pltpu.sync_copy(b1, bb)
        lane = lax.iota(jnp.int32, 16)
        zero16 = jnp.zeros((16,), jnp.int32)
        ones16 = jnp.ones((16,), jnp.float32)
        zerof16 = jnp.zeros((16,), jnp.float32)

        bs = [plsc.load_gather(bb, [zero16 + c]) for c in range(32)]
        wss = [[plsc.load_gather(wsb, [zero16 + k, zero16 + c])
                for c in range(32)] for k in range(3)]
        wns = [[plsc.load_gather(wnb, [zero16 + r, zero16 + c])
                for c in range(32)] for r in range(3)]

        for ch in range(NCHV):
            r0 = base + ch * PCV
            pltpu.sync_copy(acc_a.at[0].at[pl.ds(r0, PCV)], acc0b)
            pltpu.sync_copy(acc_a.at[1].at[pl.ds(r0, PCV)], acc1b)
            pltpu.sync_copy(v8.at[pl.ds(r0, PCV)], vbuf)
            pltpu.sync_copy(l9.at[pl.ds(r0, PCV)], lbuf)

            def group(g, carry):
                rows = g * 16 + lane
                live = jnp.where(r0 + rows < N, ones16, zerof16)

                def col(j):
                    return zero16 + j

                s = [plsc.load_gather(acc0b, [rows, col(j)])
                     + plsc.load_gather(acc1b, [rows, col(j)])
                     for j in range(4)]
                deg = s[3]
                vv = [plsc.load_gather(vbuf, [rows, col(k)]) for k in range(3)]
                rel = [s[j] - deg * vv[j] for j in range(3)]
                rot = [
                    plsc.load_gather(lbuf, [rows, col(3 * r)]) * rel[0]
                    + plsc.load_gather(lbuf, [rows, col(3 * r + 1)]) * rel[1]
                    + plsc.load_gather(lbuf, [rows, col(3 * r + 2)]) * rel[2]
                    for r in range(3)
                ]
                for c in range(32):
                    o = bs[c]
                    for k in range(3):
                        o = o + vv[k] * wss[k][c]
                    for r in range(3):
                        o = o + rot[r] * wns[r][c]
                    o = jnp.maximum(o, zerof16) * live
                    plsc.store_scatter(obuf, [rows, col(c)], o)
                return carry

            lax.fori_loop(0, PCV // 16, group, 0)
            pltpu.sync_copy(obuf, out1.at[pl.ds(r0, PCV)])

    return vert


_vert = _make_vert()


R1 = 3200
G1 = NPAD // R1


def _tc1_body(verts_ref, lrf_ref, acc_ref, w1s_ref, w1n_ref, b1_ref, out_ref):
    i = pl.program_id(0)
    s = acc_ref[0] + acc_ref[1]
    deg = s[:, 3:4]
    v = verts_ref[...]
    l9 = lrf_ref[...]
    rel = [s[:, j:j + 1] - deg * v[:, j:j + 1] for j in range(3)]
    out = b1_ref[...]
    for k in range(3):
        out = out + v[:, k:k + 1] * w1s_ref[k:k + 1, :]
    for r in range(3):
        rot_r = sum(l9[:, 3 * r + j:3 * r + j + 1] * rel[j] for j in range(3))
        out = out + rot_r * w1n_ref[r:r + 1, :]
    out = jnp.maximum(out, 0.0)
    rid = i * R1 + lax.broadcasted_iota(jnp.int32, (R1, 32), 0)
    out_ref[...] = jnp.where(rid < N, out, 0.0)


_tc1 = pl.pallas_call(
    _tc1_body,
    grid=(G1,),
    in_specs=[
        pl.BlockSpec((R1, 3), lambda i: (i, 0)),
        pl.BlockSpec((R1, 9), lambda i: (i, 0)),
        pl.BlockSpec((2, R1, 8), lambda i: (0, i, 0)),
        pl.BlockSpec((3, 32), lambda i: (0, 0)),
        pl.BlockSpec((3, 32), lambda i: (0, 0)),
        pl.BlockSpec((1, 32), lambda i: (0, 0)),
    ],
    out_specs=pl.BlockSpec((R1, 32), lambda i: (i, 0)),
    out_shape=jax.ShapeDtypeStruct((NPAD, 32), jnp.float32),
)


R2 = 1000
G2 = N // R2
BLK_PER_SEG = (N // B) // R2


def _tc2_body(out1_ref, acc_ref, w2s_ref, w2n_ref, b2_ref,
              wc1_ref, bc1_ref, wc2_ref, bc2_ref, out_ref, mref):
    i = pl.program_id(0)
    g = acc_ref[0] + acc_ref[1]
    h = jnp.dot(out1_ref[...], w2s_ref[...], preferred_element_type=jnp.float32)
    h = h + jnp.dot(g, w2n_ref[...], preferred_element_type=jnp.float32)
    out2 = jnp.maximum(h + b2_ref[...], 0.0)
    m = jnp.max(out2, axis=0, keepdims=True)
    seg = i // BLK_PER_SEG

    @pl.when(i % BLK_PER_SEG == 0)
    def _():
        mref[pl.ds(seg, 1), :] = m

    @pl.when(i % BLK_PER_SEG != 0)
    def _():
        mref[pl.ds(seg, 1), :] = jnp.maximum(mref[pl.ds(seg, 1), :], m)

    @pl.when(i == G2 - 1)
    def _():
        mf = mref[...]
        hc = jnp.dot(mf, wc1_ref[...], preferred_element_type=jnp.float32)
        hc = jnp.maximum(hc + bc1_ref[...], 0.0)
        logits = jnp.dot(hc, wc2_ref[...], preferred_element_type=jnp.float32)
        out_ref[...] = logits + bc2_ref[...]


_tc2 = pl.pallas_call(
    _tc2_body,
    grid=(G2,),
    in_specs=[
        pl.BlockSpec((R2, 32), lambda i: (i, 0)),
        pl.BlockSpec((2, R2, 32), lambda i: (0, i, 0)),
        pl.BlockSpec((32, 128), lambda i: (0, 0)),
        pl.BlockSpec((32, 128), lambda i: (0, 0)),
        pl.BlockSpec((1, 128), lambda i: (0, 0)),
        pl.BlockSpec((128, 64), lambda i: (0, 0)),
        pl.BlockSpec((1, 64), lambda i: (0, 0)),
        pl.BlockSpec((64, NUM_CLASSES), lambda i: (0, 0)),
        pl.BlockSpec((1, NUM_CLASSES), lambda i: (0, 0)),
    ],
    out_specs=pl.BlockSpec((B, NUM_CLASSES), lambda i: (0, 0)),
    out_shape=jax.ShapeDtypeStruct((B, NUM_CLASSES), jnp.float32),
    scratch_shapes=[pltpu.VMEM((B, 128), jnp.float32)],
)


def kernel(verts, edges, lrf, vert_num_list, W1_self, b1, W1_nbr,
           W2_self, b2, W2_nbr, Wc1, bc1, Wc2, bc2):
    e0 = edges[:, 0]
    e1 = edges[:, 1]
    pad = jnp.full((ED_PAD - 2 * E,), N, dtype=jnp.int32)
    src2 = jnp.concatenate([e0, e1, pad]).reshape(-1, 128)
    dst2 = jnp.concatenate([e1, e0, pad]).reshape(-1, 128)
    sdx = jnp.stack([src2, dst2], axis=1)

    verts8 = jnp.zeros((NPAD, 8), jnp.float32)
    verts8 = verts8.at[:N, :3].set(verts).at[:N, 3].set(1.0)
    zsrc8 = jnp.zeros((ZR, 8), jnp.float32)
    zsrc32 = jnp.zeros((ZR, 32), jnp.float32)

    acc_a = _segsum8(verts8, sdx, zsrc8)

    lrf9_p = jnp.zeros((NPAD, 9), jnp.float32).at[:N].set(lrf.reshape(N, 9))
    out1 = _vert(acc_a, verts8, lrf9_p, W1_self, W1_nbr, b1)

    acc_b = _segsum32(out1, sdx, zsrc32)

    logits = _tc2(out1, acc_b, W2_self, W2_nbr, b2.reshape(1, 128),
                  Wc1, bc1.reshape(1, 64), Wc2, bc2.reshape(1, NUM_CLASSES))
    return logits
